# Initial kernel scaffold; baseline (speedup 1.0000x reference)
import jax
import jax.numpy as jnp
from jax.experimental import pallas as pl


def kernel(indices, table, W, b):
    raise NotImplementedError("write your pallas kernel here")


# speedup vs baseline: 2.1540x; 2.1540x over previous
import functools

import jax
import jax.numpy as jnp
from jax import lax
from jax.experimental import pallas as pl
from jax.experimental.pallas import tpu as pltpu
from jax.experimental.pallas import tpu_sc as plsc

B, L, S = 1024, 50, 4
D_EMB, D_MODEL = 128, 512
NT = B * L
NC, NS, LANES = 2, 16, 16
NW = NC * NS
TPW = NT // NW
CHUNK = 32
NCHUNK = TPW // CHUNK
ROWS = CHUNK * S
VCH = D_EMB // LANES


def _sc_gather_sum(idx_hbm, table_hbm, feat_hbm, idx_v, rows_v, feat_v, gsem):
    w = lax.axis_index("s") * NC + lax.axis_index("c")
    pltpu.sync_copy(idx_hbm.at[w], idx_v)

    def chunk_body(c, carry):
        pltpu.async_copy(table_hbm.at[idx_v.at[c]], rows_v, gsem).wait()

        def tok_body(t, tc):
            r = 4 * t
            for h in range(VCH):
                sl = pl.ds(LANES * h, LANES)
                feat_v[t, sl] = (
                    rows_v[r, sl]
                    + rows_v[r + 1, sl]
                    + rows_v[r + 2, sl]
                    + rows_v[r + 3, sl]
                )
            return tc

        lax.fori_loop(0, CHUNK, tok_body, 0)
        pltpu.sync_copy(feat_v, feat_hbm.at[w, c])
        return carry

    lax.fori_loop(0, NCHUNK, chunk_body, 0)


_sc_call = functools.partial(
    pl.kernel,
    out_type=jax.ShapeDtypeStruct((NW, NCHUNK, CHUNK, D_EMB), jnp.float32),
    mesh=plsc.VectorSubcoreMesh(core_axis_name="c", subcore_axis_name="s"),
    scratch_types=[
        pltpu.VMEM((NCHUNK, ROWS), jnp.int32),
        pltpu.VMEM((ROWS, D_EMB), jnp.float32),
        pltpu.VMEM((CHUNK, D_EMB), jnp.float32),
        pltpu.SemaphoreType.DMA,
    ],
)(_sc_gather_sum)


def _mm_body(f_ref, w_ref, b_ref, o_ref):
    o_ref[...] = (
        jnp.dot(f_ref[...], w_ref[...], preferred_element_type=jnp.float32)
        + b_ref[...]
    )


_MM_BLK = 512


def _tc_matmul(feat, W, b2):
    return pl.pallas_call(
        _mm_body,
        grid=(NT // _MM_BLK,),
        in_specs=[
            pl.BlockSpec((_MM_BLK, D_EMB), lambda i: (i, 0)),
            pl.BlockSpec((D_EMB, D_MODEL), lambda i: (0, 0)),
            pl.BlockSpec((1, D_MODEL), lambda i: (0, 0)),
        ],
        out_specs=pl.BlockSpec((_MM_BLK, D_MODEL), lambda i: (i, 0)),
        out_shape=jax.ShapeDtypeStruct((NT, D_MODEL), jnp.float32),
    )(feat, W, b2)


def kernel(indices, table, W, b):
    idx = indices.reshape(NW, NCHUNK, ROWS).astype(jnp.int32)
    feat = _sc_call(idx, table).reshape(NT, D_EMB)
    out = _tc_matmul(feat, W, b.reshape(1, D_MODEL))
    return (out.reshape(B, L, D_MODEL), None)


# speedup vs baseline: 2.4672x; 1.1454x over previous
import functools

import jax
import jax.numpy as jnp
from jax import lax
from jax.experimental import pallas as pl
from jax.experimental.pallas import tpu as pltpu
from jax.experimental.pallas import tpu_sc as plsc

B, L, S = 1024, 50, 4
D_EMB, D_MODEL = 128, 512
NT = B * L
NC, NS, LANES = 2, 16, 16
NW = NC * NS
TPW = NT // NW
CHUNK = 32
NCHUNK = TPW // CHUNK
ROWS = CHUNK * S
VCH = D_EMB // LANES


NCHUNK_PAD = 56


def _sc_gather_sum(idx_hbm, table_hbm, feat_hbm, idx_v, rows_v, feat_v, gsem):
    w = lax.axis_index("s") * NC + lax.axis_index("c")
    pltpu.sync_copy(idx_hbm.at[w], idx_v)

    def chunk_body(c, carry):
        pltpu.async_copy(table_hbm.at[idx_v.at[c]], rows_v, gsem).wait()

        def tok_body(t, tc):
            r = 4 * t
            for h in range(VCH):
                sl = pl.ds(LANES * h, LANES)
                feat_v[t, sl] = (
                    rows_v[r, sl]
                    + rows_v[r + 1, sl]
                    + rows_v[r + 2, sl]
                    + rows_v[r + 3, sl]
                )
            return tc

        lax.fori_loop(0, CHUNK, tok_body, 0)
        pltpu.sync_copy(feat_v, feat_hbm.at[pl.ds((w * NCHUNK + c) * CHUNK, CHUNK)])
        return carry

    lax.fori_loop(0, NCHUNK, chunk_body, 0)


_sc_call = functools.partial(
    pl.kernel,
    out_type=jax.ShapeDtypeStruct((NT, D_EMB), jnp.float32),
    mesh=plsc.VectorSubcoreMesh(core_axis_name="c", subcore_axis_name="s"),
    compiler_params=pltpu.CompilerParams(use_tc_tiling_on_sc=True),
    scratch_types=[
        pltpu.VMEM((NCHUNK_PAD, ROWS), jnp.int32),
        pltpu.VMEM((ROWS, D_EMB), jnp.float32),
        pltpu.VMEM((CHUNK, D_EMB), jnp.float32),
        pltpu.SemaphoreType.DMA,
    ],
)(_sc_gather_sum)


_BB = 8
_MM_BLK = _BB * L


def _mm_body(f_ref, w_ref, b_ref, o_ref):
    m = jnp.dot(f_ref[...], w_ref[...], preferred_element_type=jnp.float32)
    o_ref[...] = m.reshape(_BB, L, D_MODEL) + b_ref[...]


def _tc_matmul(feat, W, b3):
    return pl.pallas_call(
        _mm_body,
        grid=(B // _BB,),
        in_specs=[
            pl.BlockSpec((_MM_BLK, D_EMB), lambda i: (i, 0)),
            pl.BlockSpec((D_EMB, D_MODEL), lambda i: (0, 0)),
            pl.BlockSpec((1, 1, D_MODEL), lambda i: (0, 0, 0)),
        ],
        out_specs=pl.BlockSpec((_BB, L, D_MODEL), lambda i: (i, 0, 0)),
        out_shape=jax.ShapeDtypeStruct((B, L, D_MODEL), jnp.float32),
    )(feat, W, b3)


def kernel(indices, table, W, b):
    idx = indices.reshape(NW, NCHUNK, ROWS).astype(jnp.int32)
    idx = jnp.pad(idx, ((0, 0), (0, NCHUNK_PAD - NCHUNK), (0, 0)))
    feat = _sc_call(idx, table)
    out = _tc_matmul(feat, W, b.reshape(1, 1, D_MODEL))
    return (out, None)


# speedup vs baseline: 3.4933x; 1.4159x over previous
import functools

import jax
import jax.numpy as jnp
from jax import lax
from jax.experimental import pallas as pl
from jax.experimental.pallas import tpu as pltpu
from jax.experimental.pallas import tpu_sc as plsc

B, L, S = 1024, 50, 4
D_EMB, D_MODEL = 128, 512
NC, NS, LANES = 2, 16, 16
NW = NC * NS
BPW = B // NW
ROWS = BPW * S
VCH = D_EMB // LANES


def _sc_gather_sum(idx_hbm, table_hbm, feat_hbm, idx_v, rows_v, feat_v, gsem):
    w = lax.axis_index("s") * NC + lax.axis_index("c")
    pltpu.sync_copy(idx_hbm.at[:, pl.ds(w * ROWS, ROWS)], idx_v)

    def chunk_body(l, carry):
        pltpu.async_copy(table_hbm.at[idx_v.at[l]], rows_v, gsem).wait()

        def tok_body(t, tc):
            r = 4 * t
            for h in range(VCH):
                sl = pl.ds(LANES * h, LANES)
                feat_v[t, sl] = (
                    rows_v[r, sl]
                    + rows_v[r + 1, sl]
                    + rows_v[r + 2, sl]
                    + rows_v[r + 3, sl]
                )
            return tc

        lax.fori_loop(0, BPW, tok_body, 0)
        pltpu.sync_copy(feat_v, feat_hbm.at[l, pl.ds(w * BPW, BPW)])
        return carry

    lax.fori_loop(0, L, chunk_body, 0)


_sc_call = functools.partial(
    pl.kernel,
    out_type=jax.ShapeDtypeStruct((L, B, D_EMB), jnp.float32),
    mesh=plsc.VectorSubcoreMesh(core_axis_name="c", subcore_axis_name="s"),
    compiler_params=pltpu.CompilerParams(use_tc_tiling_on_sc=True),
    scratch_types=[
        pltpu.VMEM((L, ROWS), jnp.int32),
        pltpu.VMEM((ROWS, D_EMB), jnp.float32),
        pltpu.VMEM((BPW, D_EMB), jnp.float32),
        pltpu.SemaphoreType.DMA,
    ],
)(_sc_gather_sum)


_BB = 8
_MM_BLK = L * _BB


def _mm_body(f_ref, w_ref, b_ref, o_ref):
    m = jnp.dot(
        f_ref[...].reshape(_MM_BLK, D_EMB), w_ref[...],
        preferred_element_type=jnp.float32,
    )
    o_ref[...] = m.reshape(L, _BB, D_MODEL) + b_ref[...]


def _tc_matmul(feat, W, b3):
    return pl.pallas_call(
        _mm_body,
        grid=(B // _BB,),
        in_specs=[
            pl.BlockSpec((L, _BB, D_EMB), lambda i: (0, i, 0)),
            pl.BlockSpec((D_EMB, D_MODEL), lambda i: (0, 0)),
            pl.BlockSpec((1, 1, D_MODEL), lambda i: (0, 0, 0)),
        ],
        out_specs=pl.BlockSpec((L, _BB, D_MODEL), lambda i: (0, i, 0)),
        out_shape=jax.ShapeDtypeStruct((L, B, D_MODEL), jnp.float32),
    )(feat, W, b3)


def kernel(indices, table, W, b):
    idx = jnp.transpose(indices.astype(jnp.int32), (1, 0, 2)).reshape(L, B * S)
    feat = _sc_call(idx, table)
    out = _tc_matmul(feat, W, b.reshape(1, 1, D_MODEL))
    return (jnp.transpose(out, (1, 0, 2)), None)


# speedup vs baseline: 4.7144x; 1.3495x over previous
import functools

import jax
import jax.numpy as jnp
from jax import lax
from jax.experimental import pallas as pl
from jax.experimental.pallas import tpu as pltpu
from jax.experimental.pallas import tpu_sc as plsc

B, L, S = 1024, 50, 4
D_EMB, D_MODEL = 128, 512
NC, NS, LANES = 2, 16, 16
NW = NC * NS
BPW = B // NW
ROWS = BPW * S
VCH = D_EMB // LANES


def _sc_gather_sum(idx_hbm, table_hbm, feat_hbm, idx_v,
                   rows0, rows1, feat0, feat1, gsem0, gsem1, osem0, osem1):
    w = lax.axis_index("s") * NC + lax.axis_index("c")
    rows, feat, gsem, osem = (rows0, rows1), (feat0, feat1), (gsem0, gsem1), (osem0, osem1)
    pltpu.sync_copy(idx_hbm.at[:, pl.ds(w * ROWS, ROWS)], idx_v)

    def gather(l, bi):
        return pltpu.make_async_copy(table_hbm.at[idx_v.at[l]], rows[bi], gsem[bi])

    def outcopy(l, bi):
        return pltpu.make_async_copy(
            feat[bi], feat_hbm.at[l, pl.ds(w * BPW, BPW)], osem[bi])

    gather(0, 0).start()

    def outer_body(i, carry):
        for bi in range(2):
            l = 2 * i + bi
            nb = 1 - bi

            @pl.when(l + 1 < L)
            def _():
                gather(l + 1, nb).start()

            gather(l, bi).wait()

            @pl.when(l >= 2)
            def _():
                outcopy(l - 2, bi).wait()

            def tok_body(t, tc):
                r = 4 * t
                rv = rows[bi]
                for h in range(VCH):
                    sl = pl.ds(LANES * h, LANES)
                    feat[bi][t, sl] = (
                        rv[r, sl] + rv[r + 1, sl] + rv[r + 2, sl] + rv[r + 3, sl]
                    )
                return tc

            lax.fori_loop(0, BPW, tok_body, 0)
            outcopy(l, bi).start()
        return carry

    lax.fori_loop(0, L // 2, outer_body, 0)
    outcopy(L - 2, 0).wait()
    outcopy(L - 1, 1).wait()


_sc_call = functools.partial(
    pl.kernel,
    out_type=jax.ShapeDtypeStruct((L, B, D_EMB), jnp.float32),
    mesh=plsc.VectorSubcoreMesh(core_axis_name="c", subcore_axis_name="s"),
    compiler_params=pltpu.CompilerParams(use_tc_tiling_on_sc=True),
    scratch_types=[
        pltpu.VMEM((L, ROWS), jnp.int32),
        pltpu.VMEM((ROWS, D_EMB), jnp.float32),
        pltpu.VMEM((ROWS, D_EMB), jnp.float32),
        pltpu.VMEM((BPW, D_EMB), jnp.float32),
        pltpu.VMEM((BPW, D_EMB), jnp.float32),
        pltpu.SemaphoreType.DMA,
        pltpu.SemaphoreType.DMA,
        pltpu.SemaphoreType.DMA,
        pltpu.SemaphoreType.DMA,
    ],
)(_sc_gather_sum)


_BB = 8
_MM_BLK = L * _BB


def _mm_body(f_ref, w_ref, b_ref, o_ref):
    m = jnp.dot(
        f_ref[...].reshape(_MM_BLK, D_EMB), w_ref[...],
        preferred_element_type=jnp.float32,
    )
    o_ref[...] = m.reshape(L, _BB, D_MODEL) + b_ref[...]


def _tc_matmul(feat, W, b3):
    return pl.pallas_call(
        _mm_body,
        grid=(B // _BB,),
        in_specs=[
            pl.BlockSpec((L, _BB, D_EMB), lambda i: (0, i, 0)),
            pl.BlockSpec((D_EMB, D_MODEL), lambda i: (0, 0)),
            pl.BlockSpec((1, 1, D_MODEL), lambda i: (0, 0, 0)),
        ],
        out_specs=pl.BlockSpec((L, _BB, D_MODEL), lambda i: (0, i, 0)),
        out_shape=jax.ShapeDtypeStruct((L, B, D_MODEL), jnp.float32),
    )(feat, W, b3)


def kernel(indices, table, W, b):
    idx = jnp.transpose(indices.astype(jnp.int32), (1, 0, 2)).reshape(L, B * S)
    feat = _sc_call(idx, table)
    out = _tc_matmul(feat, W, b.reshape(1, 1, D_MODEL))
    return (jnp.transpose(out, (1, 0, 2)), None)


# speedup vs baseline: 5.7455x; 1.2187x over previous
import functools

import jax
import jax.numpy as jnp
from jax import lax
from jax.experimental import pallas as pl
from jax.experimental.pallas import tpu as pltpu
from jax.experimental.pallas import tpu_sc as plsc

B, L, S = 1024, 50, 4
D_EMB, D_MODEL = 128, 512
NC, NS, LANES = 2, 16, 16
NW = NC * NS
BPW = B // NW
ROWS = BPW * S
VCH = D_EMB // LANES
NP = 2
LP = L // NP


def _sc_gather_sum(lp0, idx_hbm, table_hbm, feat_hbm, idx_v,
                   rows0, rows1, feat0, feat1, gsem0, gsem1, osem0, osem1):
    w = lax.axis_index("s") * NC + lax.axis_index("c")
    rows, feat, gsem, osem = (rows0, rows1), (feat0, feat1), (gsem0, gsem1), (osem0, osem1)
    pltpu.sync_copy(idx_hbm.at[:, pl.ds(w * ROWS, ROWS)], idx_v)

    def gather(l, bi):
        return pltpu.make_async_copy(
            table_hbm.at[idx_v.at[lp0 + l]], rows[bi], gsem[bi])

    def outcopy(l, bi):
        return pltpu.make_async_copy(
            feat[bi], feat_hbm.at[l, pl.ds(w * BPW, BPW)], osem[bi])

    def chunk(l, bi):
        gather(l, bi).wait()

        @pl.when(l >= 2)
        def _():
            outcopy(l - 2, bi).wait()

        def tok_body(t, tc):
            r = 4 * t
            rv = rows[bi]
            for h in range(VCH):
                sl = pl.ds(LANES * h, LANES)
                feat[bi][t, sl] = (
                    rv[r, sl] + rv[r + 1, sl] + rv[r + 2, sl] + rv[r + 3, sl]
                )
            return tc

        lax.fori_loop(0, BPW, tok_body, 0)
        outcopy(l, bi).start()

    gather(0, 0).start()

    def outer_body(i, carry):
        for bi in range(2):
            l = 2 * i + bi

            @pl.when(l + 1 < LP)
            def _():
                gather(l + 1, 1 - bi).start()

            chunk(l, bi)
        return carry

    lax.fori_loop(0, LP // 2, outer_body, 0)
    if LP % 2:
        chunk(LP - 1, (LP - 1) % 2)
    outcopy(LP - 2, (LP - 2) % 2).wait()
    outcopy(LP - 1, (LP - 1) % 2).wait()


def _make_sc_call(lp0):
    return functools.partial(
        pl.kernel,
        out_type=jax.ShapeDtypeStruct((LP, B, D_EMB), jnp.float32),
        mesh=plsc.VectorSubcoreMesh(core_axis_name="c", subcore_axis_name="s"),
        compiler_params=pltpu.CompilerParams(use_tc_tiling_on_sc=True),
        scratch_types=[
            pltpu.VMEM((L, ROWS), jnp.int32),
            pltpu.VMEM((ROWS, D_EMB), jnp.float32),
            pltpu.VMEM((ROWS, D_EMB), jnp.float32),
            pltpu.VMEM((BPW, D_EMB), jnp.float32),
            pltpu.VMEM((BPW, D_EMB), jnp.float32),
            pltpu.SemaphoreType.DMA,
            pltpu.SemaphoreType.DMA,
            pltpu.SemaphoreType.DMA,
            pltpu.SemaphoreType.DMA,
        ],
    )(functools.partial(_sc_gather_sum, lp0))


_sc_calls = [_make_sc_call(p * LP) for p in range(NP)]


_BB = 16
_MM_BLK = LP * _BB


def _mm_compute(f_ref, w_ref, b_ref, o_ref):
    m = jnp.dot(
        f_ref[...].reshape(_MM_BLK, D_EMB), w_ref[...],
        preferred_element_type=jnp.float32,
    )
    o_ref[...] = m.reshape(LP, _BB, D_MODEL) + b_ref[...]


def _mm_body0(f_ref, w_ref, b_ref, o_ref):
    _mm_compute(f_ref, w_ref, b_ref, o_ref)


def _mm_body1(f_ref, w_ref, b_ref, prev_ref, o_ref):
    del prev_ref
    _mm_compute(f_ref, w_ref, b_ref, o_ref)


def _tc_matmul(feats, W, b3):
    f_spec = pl.BlockSpec((LP, _BB, D_EMB), lambda i: (0, i, 0))
    w_spec = pl.BlockSpec((D_EMB, D_MODEL), lambda i: (0, 0))
    b_spec = pl.BlockSpec((1, 1, D_MODEL), lambda i: (0, 0, 0))
    out_shape = jax.ShapeDtypeStruct((L, B, D_MODEL), jnp.float32)

    out0 = pl.pallas_call(
        _mm_body0,
        grid=(B // _BB,),
        in_specs=[f_spec, w_spec, b_spec],
        out_specs=pl.BlockSpec((LP, _BB, D_MODEL), lambda i: (0, i, 0)),
        out_shape=out_shape,
    )(feats[0], W, b3)
    return pl.pallas_call(
        _mm_body1,
        grid=(B // _BB,),
        in_specs=[f_spec, w_spec, b_spec,
                  pl.BlockSpec(memory_space=pl.ANY)],
        out_specs=pl.BlockSpec((LP, _BB, D_MODEL), lambda i: (1, i, 0)),
        out_shape=out_shape,
        input_output_aliases={3: 0},
    )(feats[1], W, b3, out0)


def kernel(indices, table, W, b):
    idx = jnp.transpose(indices.astype(jnp.int32), (1, 0, 2)).reshape(L, B * S)
    feats = [call(idx, table) for call in _sc_calls]
    out = _tc_matmul(feats, W, b.reshape(1, 1, D_MODEL))
    return (jnp.transpose(out, (1, 0, 2)), None)


# speedup vs baseline: 5.8311x; 1.0149x over previous
import functools

import jax
import jax.numpy as jnp
from jax import lax
from jax.experimental import pallas as pl
from jax.experimental.pallas import tpu as pltpu
from jax.experimental.pallas import tpu_sc as plsc

B, L, S = 1024, 50, 4
D_EMB, D_MODEL = 128, 512
NC, NS, LANES = 2, 16, 16
NW = NC * NS
BPW = B // NW
ROWS = BPW * S
VCH = D_EMB // LANES
NP = 5
LP = L // NP


def _sc_gather_sum(lp0, idx_hbm, table_hbm, feat_hbm, idx_v,
                   rows0, rows1, feat0, feat1, gsem0, gsem1, osem0, osem1):
    w = lax.axis_index("s") * NC + lax.axis_index("c")
    rows, feat, gsem, osem = (rows0, rows1), (feat0, feat1), (gsem0, gsem1), (osem0, osem1)
    pltpu.sync_copy(idx_hbm.at[:, pl.ds(w * ROWS, ROWS)], idx_v)

    def gather(l, bi):
        return pltpu.make_async_copy(
            table_hbm.at[idx_v.at[lp0 + l]], rows[bi], gsem[bi])

    def outcopy(l, bi):
        return pltpu.make_async_copy(
            feat[bi], feat_hbm.at[l, pl.ds(w * BPW, BPW)], osem[bi])

    def chunk(l, bi):
        gather(l, bi).wait()

        @pl.when(l >= 2)
        def _():
            outcopy(l - 2, bi).wait()

        def tok_body(t, tc):
            r = 4 * t
            rv = rows[bi]
            for h in range(VCH):
                sl = pl.ds(LANES * h, LANES)
                feat[bi][t, sl] = (
                    rv[r, sl] + rv[r + 1, sl] + rv[r + 2, sl] + rv[r + 3, sl]
                )
            return tc

        lax.fori_loop(0, BPW, tok_body, 0)
        outcopy(l, bi).start()

    gather(0, 0).start()

    def outer_body(i, carry):
        for bi in range(2):
            l = 2 * i + bi

            @pl.when(l + 1 < LP)
            def _():
                gather(l + 1, 1 - bi).start()

            chunk(l, bi)
        return carry

    lax.fori_loop(0, LP // 2, outer_body, 0)
    if LP % 2:
        chunk(LP - 1, (LP - 1) % 2)
    outcopy(LP - 2, (LP - 2) % 2).wait()
    outcopy(LP - 1, (LP - 1) % 2).wait()


def _make_sc_call(lp0):
    return functools.partial(
        pl.kernel,
        out_type=jax.ShapeDtypeStruct((LP, B, D_EMB), jnp.float32),
        mesh=plsc.VectorSubcoreMesh(core_axis_name="c", subcore_axis_name="s"),
        compiler_params=pltpu.CompilerParams(use_tc_tiling_on_sc=True),
        scratch_types=[
            pltpu.VMEM((L, ROWS), jnp.int32),
            pltpu.VMEM((ROWS, D_EMB), jnp.float32),
            pltpu.VMEM((ROWS, D_EMB), jnp.float32),
            pltpu.VMEM((BPW, D_EMB), jnp.float32),
            pltpu.VMEM((BPW, D_EMB), jnp.float32),
            pltpu.SemaphoreType.DMA,
            pltpu.SemaphoreType.DMA,
            pltpu.SemaphoreType.DMA,
            pltpu.SemaphoreType.DMA,
        ],
    )(functools.partial(_sc_gather_sum, lp0))


_sc_calls = [_make_sc_call(p * LP) for p in range(NP)]


_BB = 32
_MM_BLK = LP * _BB


def _mm_compute(f_ref, w_ref, b_ref, o_ref):
    m = jnp.dot(
        f_ref[...].reshape(_MM_BLK, D_EMB), w_ref[...],
        preferred_element_type=jnp.float32,
    )
    o_ref[...] = m.reshape(LP, _BB, D_MODEL) + b_ref[...]


def _mm_body0(f_ref, w_ref, b_ref, o_ref):
    _mm_compute(f_ref, w_ref, b_ref, o_ref)


def _mm_body1(f_ref, w_ref, b_ref, prev_ref, o_ref):
    del prev_ref
    _mm_compute(f_ref, w_ref, b_ref, o_ref)


def _make_out_index_map(p):
    return lambda i: (p, i, 0)


def _tc_matmul(feats, W, b3):
    f_spec = pl.BlockSpec((LP, _BB, D_EMB), lambda i: (0, i, 0))
    w_spec = pl.BlockSpec((D_EMB, D_MODEL), lambda i: (0, 0))
    b_spec = pl.BlockSpec((1, 1, D_MODEL), lambda i: (0, 0, 0))
    out_shape = jax.ShapeDtypeStruct((L, B, D_MODEL), jnp.float32)

    out = pl.pallas_call(
        _mm_body0,
        grid=(B // _BB,),
        in_specs=[f_spec, w_spec, b_spec],
        out_specs=pl.BlockSpec((LP, _BB, D_MODEL), _make_out_index_map(0)),
        out_shape=out_shape,
    )(feats[0], W, b3)
    for p in range(1, NP):
        out = pl.pallas_call(
            _mm_body1,
            grid=(B // _BB,),
            in_specs=[f_spec, w_spec, b_spec,
                      pl.BlockSpec(memory_space=pl.ANY)],
            out_specs=pl.BlockSpec((LP, _BB, D_MODEL), _make_out_index_map(p)),
            out_shape=out_shape,
            input_output_aliases={3: 0},
        )(feats[p], W, b3, out)
    return out


def kernel(indices, table, W, b):
    idx = jnp.transpose(indices.astype(jnp.int32), (1, 0, 2)).reshape(L, B * S)
    feats = [call(idx, table) for call in _sc_calls]
    out = _tc_matmul(feats, W, b.reshape(1, 1, D_MODEL))
    return (jnp.transpose(out, (1, 0, 2)), None)
